# speedup vs baseline: 3.4099x; 3.4099x over previous
import jax
import jax.numpy as jnp
from jax.experimental import pallas as pl


_BLOCK = 512


def _bcast_copy_kernel(table_ref, out_ref):
    out_ref[0] = table_ref[...]


def kernel(x, table):
    batch, seq_len = x.shape
    d_model = table.shape[1]
    n_blocks = seq_len // _BLOCK
    return pl.pallas_call(
        _bcast_copy_kernel,
        grid=(n_blocks, batch),
        in_specs=[
            pl.BlockSpec((_BLOCK, d_model), lambda j, b: (j, 0)),
        ],
        out_specs=pl.BlockSpec((1, _BLOCK, d_model), lambda j, b: (b, j, 0)),
        out_shape=jax.ShapeDtypeStruct((batch, seq_len, d_model), table.dtype),
    )(table)


# speedup vs baseline: 5.0406x; 1.4782x over previous
import jax
import jax.numpy as jnp
from jax.experimental import pallas as pl


_BLOCK = 512


def _bcast_kernel(table_ref, out_ref):
    for b in range(out_ref.shape[0]):
        out_ref[b] = table_ref[...]


def kernel(x, table):
    batch, seq_len = x.shape
    d_model = table.shape[1]
    n_blocks = seq_len // _BLOCK
    return pl.pallas_call(
        _bcast_kernel,
        grid=(n_blocks,),
        in_specs=[
            pl.BlockSpec((_BLOCK, d_model), lambda j: (j, 0)),
        ],
        out_specs=pl.BlockSpec((batch, _BLOCK, d_model), lambda j: (0, j, 0)),
        out_shape=jax.ShapeDtypeStruct((batch, seq_len, d_model), table.dtype),
    )(table)
